# baseline (speedup 1.0000x reference)
import jax
import jax.numpy as jnp
from jax import lax
from jax.experimental import pallas as pl
from jax.experimental.pallas import tpu as pltpu

F32 = jnp.float32

_FB = 1024
_FQ = 1024
_DB = 256


def _body(wdecup_ref, flatcol_ref, wenc_ref, conn_ref, acts_ref, bdecup_ref,
          wdecd_ref, bdecd_ref, recon_ref, vals_ref, idx_ref,
          aup_ref, approxs_ref, feats_ref):
    i = pl.program_id(0)
    J = flatcol_ref.shape[0]
    B = acts_ref.shape[0]
    K_up = J // B
    F_down = approxs_ref.shape[1]
    K_down = vals_ref.shape[1]
    nq = 4
    nfb = 4

    @pl.when(i < nq)
    def _build_aup():
        base = i * _FQ
        onehot = (lax.broadcasted_iota(jnp.int32, (J, _FQ), 1) + base
                  == flatcol_ref[:, :]).astype(F32)
        part = lax.dot_general(
            onehot, wdecup_ref[:, :], (((1,), (1,)), ((), ())),
            preferred_element_type=F32)

        @pl.when(i == 0)
        def _():
            aup_ref[:, :] = part

        @pl.when(i > 0)
        def _():
            aup_ref[:, :] = aup_ref[:, :] + part

    @pl.when((i >= nq) & (i < nq + nfb))
    def _block():
        C, Fb = conn_ref.shape
        virt_t = lax.dot_general(
            aup_ref[:, :], wenc_ref[:, :], (((1,), (1,)), ((), ())),
            preferred_element_type=F32)
        cnt = jnp.zeros((J, Fb), F32)
        for c in range(C):
            cnt = cnt + (conn_ref[c:c + 1, :] == flatcol_ref[:, :]).astype(F32)
        pmat = virt_t * cnt
        pruned_t = jnp.sum(pmat.reshape(B, K_up, Fb), axis=1)
        contrib_t = lax.dot_general(
            acts_ref[:, :], wenc_ref[:, :], (((1,), (1,)), ((), ())),
            preferred_element_type=F32)
        bcontrib_t = lax.dot_general(
            bdecup_ref[:, :], wenc_ref[:, :], (((1,), (1,)), ((), ())),
            preferred_element_type=F32)
        approxs_ref[:, pl.ds((i - nq) * _FB, _FB)] = (
            contrib_t + (pruned_t + bcontrib_t))

    @pl.when(i == nq + nfb)
    def _topk():
        work = approxs_ref[:, :]
        lane = lax.broadcasted_iota(jnp.int32, (B, F_down), 1)
        feats = jnp.zeros((B, F_down), F32)
        neg = jnp.float32(-jnp.inf)
        for j in range(K_down):
            m = jnp.max(work, axis=1, keepdims=True)
            cand = jnp.where(work == m, lane, F_down)
            sel = jnp.min(cand, axis=1, keepdims=True)
            chosen = lane == sel
            feats = feats + jnp.where(chosen, work, 0.0)
            vals_ref[:, j:j + 1] = m
            idx_ref[:, j:j + 1] = sel
            work = jnp.where(chosen, neg, work)
        feats_ref[:, :] = feats

    @pl.when(i >= nq + nfb)
    def _decode():
        recon_ref[:, :] = lax.dot_general(
            feats_ref[:, :], wdecd_ref[:, :], (((1,), (1,)), ((), ())),
            preferred_element_type=F32) + bdecd_ref[:, :]


def kernel(initial_acts, up_indices, up_vals, connections, W_enc_down,
           W_dec_down, W_dec_up, b_dec_up, b_dec_down):
    del up_vals
    B, D = initial_acts.shape
    F_down, C = connections.shape
    _, F_up = W_dec_up.shape
    K_up = up_indices.shape[1]
    J = B * K_up
    K_down = 32

    flat_col = up_indices.reshape(J, 1).astype(jnp.int32)

    nq = F_up // _FQ
    nfb = F_down // _FB
    nd = D // _DB
    grid = (nq + nfb + nd,)

    qidx = lambda i: (0, jnp.minimum(i, nq - 1))
    fidx = lambda i: (jnp.clip(i - nq, 0, nfb - 1), 0)
    fidx_t = lambda i: (0, jnp.clip(i - nq, 0, nfb - 1))
    didx = lambda i: (jnp.clip(i - nq - nfb, 0, nd - 1), 0)
    didx_t = lambda i: (0, jnp.clip(i - nq - nfb, 0, nd - 1))

    recon, vals, idx = pl.pallas_call(
        _body,
        grid=grid,
        in_specs=[
            pl.BlockSpec((D, _FQ), qidx),
            pl.BlockSpec((J, 1), lambda i: (0, 0)),
            pl.BlockSpec((_FB, D), fidx),
            pl.BlockSpec((C, _FB), fidx_t),
            pl.BlockSpec((B, D), lambda i: (0, 0)),
            pl.BlockSpec((1, D), lambda i: (0, 0)),
            pl.BlockSpec((_DB, F_down), didx),
            pl.BlockSpec((1, _DB), didx_t),
        ],
        out_specs=[
            pl.BlockSpec((B, _DB), didx_t),
            pl.BlockSpec((B, K_down), lambda i: (0, 0)),
            pl.BlockSpec((B, K_down), lambda i: (0, 0)),
        ],
        out_shape=[
            jax.ShapeDtypeStruct((B, D), F32),
            jax.ShapeDtypeStruct((B, K_down), F32),
            jax.ShapeDtypeStruct((B, K_down), jnp.int32),
        ],
        scratch_shapes=[
            pltpu.VMEM((J, D), F32),
            pltpu.VMEM((B, F_down), F32),
            pltpu.VMEM((B, F_down), F32),
        ],
    )(W_dec_up, flat_col, W_enc_down, connections.T, initial_acts,
      b_dec_up.reshape(1, D), W_dec_down, b_dec_down.reshape(1, D))

    return recon, vals, idx


# speedup vs baseline: 1.0977x; 1.0977x over previous
import jax
import jax.numpy as jnp
from jax import lax
from jax.experimental import pallas as pl
from jax.experimental.pallas import tpu as pltpu

F32 = jnp.float32

_FB = 1024
_FQ = 1024
_DB = 256


def _body(wdecup_ref, flatcol_ref, wenc_ref, conn_ref, acts_ref, bdecup_ref,
          wdecd_ref, bdecd_ref, recon_ref, vals_ref, idx_ref,
          aup_ref, approxs_ref, feats_ref):
    i = pl.program_id(0)
    J = flatcol_ref.shape[0]
    B = acts_ref.shape[0]
    K_up = J // B
    F_down = approxs_ref.shape[1]
    K_down = vals_ref.shape[1]
    nq = 4
    nfb = 4

    @pl.when(i < nq)
    def _build_aup():
        base = i * _FQ
        onehot = (lax.broadcasted_iota(jnp.int32, (J, _FQ), 1) + base
                  == flatcol_ref[:, :]).astype(F32)
        part = lax.dot_general(
            onehot, wdecup_ref[:, :], (((1,), (1,)), ((), ())),
            preferred_element_type=F32)

        @pl.when(i == 0)
        def _():
            aup_ref[:, :] = part

        @pl.when(i > 0)
        def _():
            aup_ref[:, :] = aup_ref[:, :] + part

    @pl.when((i >= nq) & (i < nq + nfb))
    def _block():
        C, Fb = conn_ref.shape
        virt_t = lax.dot_general(
            aup_ref[:, :], wenc_ref[:, :], (((1,), (1,)), ((), ())),
            preferred_element_type=F32)
        fc16 = flatcol_ref[:, :].astype(jnp.int16)
        cnt = jnp.zeros((J, Fb), jnp.bfloat16)
        one = jnp.ones((), jnp.bfloat16)
        zero = jnp.zeros((), jnp.bfloat16)
        for c in range(C):
            cnt = cnt + jnp.where(conn_ref[c:c + 1, :] == fc16, one, zero)
        pmat = virt_t * cnt.astype(F32)
        pruned_t = jnp.sum(pmat.reshape(B, K_up, Fb), axis=1)
        contrib_t = lax.dot_general(
            acts_ref[:, :], wenc_ref[:, :], (((1,), (1,)), ((), ())),
            preferred_element_type=F32)
        bcontrib_t = lax.dot_general(
            bdecup_ref[:, :], wenc_ref[:, :], (((1,), (1,)), ((), ())),
            preferred_element_type=F32)
        approxs_ref[:, pl.ds((i - nq) * _FB, _FB)] = (
            contrib_t + (pruned_t + bcontrib_t))

    @pl.when(i == nq + nfb)
    def _topk():
        work = approxs_ref[:, :]
        lane = lax.broadcasted_iota(jnp.int32, (B, F_down), 1)
        feats = jnp.zeros((B, F_down), F32)
        neg = jnp.float32(-jnp.inf)
        for j in range(K_down):
            m = jnp.max(work, axis=1, keepdims=True)
            cand = jnp.where(work == m, lane, F_down)
            sel = jnp.min(cand, axis=1, keepdims=True)
            chosen = lane == sel
            feats = feats + jnp.where(chosen, work, 0.0)
            vals_ref[:, j:j + 1] = m
            idx_ref[:, j:j + 1] = sel
            work = jnp.where(chosen, neg, work)
        feats_ref[:, :] = feats

    @pl.when(i >= nq + nfb)
    def _decode():
        recon_ref[:, :] = lax.dot_general(
            feats_ref[:, :], wdecd_ref[:, :], (((1,), (1,)), ((), ())),
            preferred_element_type=F32) + bdecd_ref[:, :]


def kernel(initial_acts, up_indices, up_vals, connections, W_enc_down,
           W_dec_down, W_dec_up, b_dec_up, b_dec_down):
    del up_vals
    B, D = initial_acts.shape
    F_down, C = connections.shape
    _, F_up = W_dec_up.shape
    K_up = up_indices.shape[1]
    J = B * K_up
    K_down = 32

    flat_col = up_indices.reshape(J, 1).astype(jnp.int32)

    nq = F_up // _FQ
    nfb = F_down // _FB
    nd = D // _DB
    grid = (nq + nfb + nd,)

    qidx = lambda i: (0, jnp.minimum(i, nq - 1))
    fidx = lambda i: (jnp.clip(i - nq, 0, nfb - 1), 0)
    fidx_t = lambda i: (0, jnp.clip(i - nq, 0, nfb - 1))
    didx = lambda i: (jnp.clip(i - nq - nfb, 0, nd - 1), 0)
    didx_t = lambda i: (0, jnp.clip(i - nq - nfb, 0, nd - 1))

    recon, vals, idx = pl.pallas_call(
        _body,
        grid=grid,
        in_specs=[
            pl.BlockSpec((D, _FQ), qidx),
            pl.BlockSpec((J, 1), lambda i: (0, 0)),
            pl.BlockSpec((_FB, D), fidx),
            pl.BlockSpec((C, _FB), fidx_t),
            pl.BlockSpec((B, D), lambda i: (0, 0)),
            pl.BlockSpec((1, D), lambda i: (0, 0)),
            pl.BlockSpec((_DB, F_down), didx),
            pl.BlockSpec((1, _DB), didx_t),
        ],
        out_specs=[
            pl.BlockSpec((B, _DB), didx_t),
            pl.BlockSpec((B, K_down), lambda i: (0, 0)),
            pl.BlockSpec((B, K_down), lambda i: (0, 0)),
        ],
        out_shape=[
            jax.ShapeDtypeStruct((B, D), F32),
            jax.ShapeDtypeStruct((B, K_down), F32),
            jax.ShapeDtypeStruct((B, K_down), jnp.int32),
        ],
        scratch_shapes=[
            pltpu.VMEM((J, D), F32),
            pltpu.VMEM((B, F_down), F32),
            pltpu.VMEM((B, F_down), F32),
        ],
    )(W_dec_up, flat_col, W_enc_down, connections.T.astype(jnp.int16),
      initial_acts,
      b_dec_up.reshape(1, D), W_dec_down, b_dec_down.reshape(1, D))

    return recon, vals, idx
